# baseline (speedup 1.0000x reference)
import functools

import jax
import jax.numpy as jnp
from jax import lax
from jax.experimental import pallas as pl
from jax.experimental.pallas import tpu as pltpu
from jax.experimental.pallas import tpu_sc as plsc

_N = 100000
_E = 3200000
_LAMBDA_CONT = 0.1
_LAMBDA_MOM = 0.01
_REYNOLDS = 1000000.0

_NW = 32
_SUB = 128
_NCHUNK = _E // _SUB
_CPW = 781
_KB = 71
_NB = 11
_NP = 100096
_RPT = _NP // 16
_TRB = 272


def _compute_chunk(rrows, crows, obuf):
    lane = lax.iota(jnp.int32, 16)
    for g in range(_SUB // 16):
        rid = lane + (g * 16)

        def ld(ref, k):
            return plsc.load_gather(
                ref, [rid, jnp.full((16,), k, jnp.int32)])

        rvx, rvy, rvz = ld(rrows, 0), ld(rrows, 1), ld(rrows, 2)
        rp = ld(rrows, 3)
        rpx, rpy, rpz = ld(rrows, 4), ld(rrows, 5), ld(rrows, 6)
        cvx, cvy, cvz = ld(crows, 0), ld(crows, 1), ld(crows, 2)
        cp = ld(crows, 3)
        cpx, cpy, cpz = ld(crows, 4), ld(crows, 5), ld(crows, 6)

        dx = cpx - rpx
        dy = cpy - rpy
        dz = cpz - rpz
        vdx = cvx - rvx
        vdy = cvy - rvy
        vdz = cvz - rvz
        pd = cp - rp

        d2 = dx * dx + dy * dy + dz * dz
        d2c = jnp.maximum(d2, 1.1754944e-38)
        bits = plsc.bitcast(d2c, jnp.int32)
        y = plsc.bitcast(
            jnp.full((16,), 0x5F3759DF, jnp.int32) - (bits >> 1),
            jnp.float32)
        h = 0.5 * d2c
        y = y * (1.5 - (h * y) * y)
        y = y * (1.5 - (h * y) * y)
        y = y * (1.5 - (h * y) * y)
        inv_dist = y - 1e-8 * (y * y)
        inv_d2e = 1.0 / (d2 + 1e-8)

        vg = (vdx * dx + vdy * dy + vdz * dz) * inv_dist
        wdx = vdx * inv_d2e
        wdy = vdy * inv_d2e
        wdz = vdz * inv_d2e
        pt = pd * inv_dist * inv_dist
        pgx = pt * dx
        pgy = pt * dy
        pgz = pt * dz
        ones = jnp.full((16,), 1.0, jnp.float32)

        for k, val in enumerate((vg, wdx, wdy, wdz, pgx, pgy, pgz,
                                 ones)):
            plsc.store_scatter(
                obuf, [rid, jnp.full((16,), k, jnp.int32)], val)


def _edge_body(tbl, row_hbm, col_hbm, zrows, out_hbm,
               acc_sh, tbl_sh, ridx2_v, cidx2_v,
               rrows_a, crows_a, obuf_a, rrows_b, crows_b, obuf_b,
               rrows_c, crows_c, obuf_c,
               tin_v, tout2_v,
               gsem_a, gsem_b, gsem_c,
               ssem_a, ssem_b, ssem_c, tsem_a, tsem_b):
    cid = lax.axis_index("c")
    sid = lax.axis_index("s")
    wid = sid * 2 + cid
    wstart = _CPW * wid + jnp.minimum(wid, 8)

    pltpu.sync_copy(zrows, acc_sh.at[pl.ds(sid * _RPT, _RPT)])
    pltpu.sync_copy(tbl.at[pl.ds(sid * (_N // 16), _N // 16)],
                    tbl_sh.at[pl.ds(sid * (_N // 16), _N // 16)])
    plsc.subcore_barrier()

    def ridx(j):
        return ridx2_v.at[pl.ds(j * _SUB, _SUB)]

    def cidx(j):
        return cidx2_v.at[pl.ds(j * _SUB, _SUB)]

    def fire_gather(j, rrows, crows, gsem):
        gr = pltpu.async_copy(tbl_sh.at[ridx(j)], rrows, gsem)
        gc = pltpu.async_copy(tbl_sh.at[cidx(j)], crows, gsem)
        return gr, gc

    def wait_gather(j, rrows, crows, gsem):
        pltpu.make_async_copy(tbl_sh.at[ridx(j)], rrows, gsem).wait()
        pltpu.make_async_copy(tbl_sh.at[cidx(j)], crows, gsem).wait()

    def fire_scatter(j, obuf, ssem):
        return pltpu.async_copy(obuf, acc_sh.at[ridx(j)], ssem,
                                add=True)

    def wait_scatter(obuf, ssem):
        pltpu.make_async_copy(obuf, acc_sh.at[ridx(0)], ssem).wait()

    slots = ((rrows_a, crows_a, obuf_a, gsem_a, ssem_a),
             (rrows_b, crows_b, obuf_b, gsem_b, ssem_b),
             (rrows_c, crows_c, obuf_c, gsem_c, ssem_c))

    def block(b, _):
        start = (wstart + b * _KB) * _SUB
        pltpu.sync_copy(row_hbm.at[pl.ds(start, _KB * _SUB)], ridx2_v)
        pltpu.sync_copy(col_hbm.at[pl.ds(start, _KB * _SUB)], cidx2_v)
        for s in range(3):
            rr, cr, ob, gs, ss = slots[s]
            fire_gather(s, rr, cr, gs)

        def triple(t, _):
            j0 = 3 * t
            for s in range(3):
                rr, cr, ob, gs, ss = slots[s]
                wait_gather(j0 + s, rr, cr, gs)

                @pl.when(t > 0)
                def _():
                    wait_scatter(ob, ss)

                _compute_chunk(rr, cr, ob)
                fire_scatter(j0 + s, ob, ss)

                @pl.when(j0 + s + 3 < _KB)
                def _():
                    fire_gather(j0 + s + 3, rr, cr, gs)
            return ()

        lax.fori_loop(0, (_KB - 2) // 3, triple, ())

        fin = []
        for s in range(2):
            rr, cr, ob, gs, ss = slots[s]
            j = _KB - 2 + s
            wait_gather(j, rr, cr, gs)
            wait_scatter(ob, ss)
            _compute_chunk(rr, cr, ob)
            fin.append(fire_scatter(j, ob, ss))
        wait_scatter(obuf_c, ssem_c)
        for f in fin:
            f.wait()
        return ()

    lax.fori_loop(0, _NB, block, ())

    @pl.when(wid < 8)
    def _():
        extra = (wstart + _CPW) * _SUB
        pltpu.sync_copy(row_hbm.at[pl.ds(extra, _SUB)],
                        ridx2_v.at[pl.ds(0, _SUB)])
        pltpu.sync_copy(col_hbm.at[pl.ds(extra, _SUB)],
                        cidx2_v.at[pl.ds(0, _SUB)])
        gr, gc = fire_gather(0, rrows_a, crows_a, gsem_a)
        gr.wait()
        gc.wait()
        _compute_chunk(rrows_a, crows_a, obuf_a)
        pltpu.sync_copy(obuf_a, acc_sh.at[ridx(0)], add=True)

    plsc.subcore_barrier()

    lane = lax.iota(jnp.int32, 16)

    def trblock(t, _):
        local = t * _TRB
        pltpu.sync_copy(acc_sh.at[pl.ds(sid * _RPT + local, _TRB)], tin_v)
        for k in range(8):
            buf = tout2_v.at[k % 2]
            tsem = tsem_a if k % 2 == 0 else tsem_b
            if k >= 2:
                pltpu.make_async_copy(
                    buf, out_hbm.at[cid, sid, k - 2, pl.ds(local, _TRB)],
                    tsem).wait()
            kidx = jnp.full((16,), k, jnp.int32)
            for g in range(_TRB // 16):
                v = plsc.load_gather(tin_v, [lane + (g * 16), kidx])
                buf[pl.ds(g * 16, 16)] = v
            pltpu.async_copy(
                buf, out_hbm.at[cid, sid, k, pl.ds(local, _TRB)], tsem)
        for k in (6, 7):
            buf = tout2_v.at[k % 2]
            tsem = tsem_a if k % 2 == 0 else tsem_b
            pltpu.make_async_copy(
                buf, out_hbm.at[cid, sid, k, pl.ds(local, _TRB)],
                tsem).wait()
        return ()

    lax.fori_loop(0, _RPT // _TRB, trblock, ())


def _finalize_body(acc_ref, pred_ref, tgt_ref, out_ref):
    c_sum = jnp.float32(0.0)
    mom = jnp.float32(0.0)
    for t in range(16):
        a = acc_ref[0, t] + acc_ref[1, t]
        cnt = jnp.maximum(a[7:8, :], 1.0)
        inv_cnt = 1.0 / cnt
        div = a[0:1, :] * inv_cnt
        c_sum = c_sum + jnp.sum(div * div)
        for k in range(3):
            res = (a[1 + k:2 + k, :] * (1.0 / _REYNOLDS)
                   + a[4 + k:5 + k, :]) * inv_cnt
            mom = mom + jnp.sum(res * res)

    d = pred_ref[...] - tgt_ref[...]
    dsq = d * d
    col = lax.broadcasted_iota(jnp.int32, dsq.shape, 1)
    is_p = (col % 4) == 3
    p_sum = jnp.sum(jnp.where(is_p, dsq, 0.0))
    v_sum = jnp.sum(jnp.where(is_p, 0.0, dsq))

    total = (v_sum / (3.0 * _N) + p_sum / _N
             + _LAMBDA_CONT * (c_sum / _N)
             + _LAMBDA_MOM * (mom / (3.0 * _N)))
    out_ref[...] = jnp.reshape(total, (1, 1))


@jax.jit
def kernel(pred, target, edge_index, pos):
    tbl = jnp.concatenate(
        [pred, pos, jnp.zeros((_N, 1), jnp.float32)], axis=1)
    row1 = edge_index[0]
    col1 = edge_index[1]
    zrows = jnp.zeros((_RPT, 8), jnp.float32)

    edge_fn = pl.kernel(
        _edge_body,
        out_type=jax.ShapeDtypeStruct((2, 16, 8, _RPT), jnp.float32),
        mesh=plsc.VectorSubcoreMesh(core_axis_name="c",
                                    subcore_axis_name="s"),
        scratch_types=[
            pltpu.VMEM_SHARED((_NP, 8), jnp.float32),
            pltpu.VMEM_SHARED((_N, 8), jnp.float32),
            pltpu.VMEM((_KB * _SUB,), jnp.int32),
            pltpu.VMEM((_KB * _SUB,), jnp.int32),
            pltpu.VMEM((_SUB, 8), jnp.float32),
            pltpu.VMEM((_SUB, 8), jnp.float32),
            pltpu.VMEM((_SUB, 8), jnp.float32),
            pltpu.VMEM((_SUB, 8), jnp.float32),
            pltpu.VMEM((_SUB, 8), jnp.float32),
            pltpu.VMEM((_SUB, 8), jnp.float32),
            pltpu.VMEM((_SUB, 8), jnp.float32),
            pltpu.VMEM((_SUB, 8), jnp.float32),
            pltpu.VMEM((_SUB, 8), jnp.float32),
            pltpu.VMEM((_TRB, 8), jnp.float32),
            pltpu.VMEM((2, _TRB), jnp.float32),
            pltpu.SemaphoreType.DMA,
            pltpu.SemaphoreType.DMA,
            pltpu.SemaphoreType.DMA,
            pltpu.SemaphoreType.DMA,
            pltpu.SemaphoreType.DMA,
            pltpu.SemaphoreType.DMA,
            pltpu.SemaphoreType.DMA,
            pltpu.SemaphoreType.DMA,
        ],
        compiler_params=pltpu.CompilerParams(
            needs_layout_passes=False,
            use_tc_tiling_on_sc=False,
            internal_scratch_in_bytes=1 << 20,
        ),
    )
    parts = edge_fn(tbl, row1, col1, zrows)

    pred_r = pred.reshape(3125, 128)
    tgt_r = target.reshape(3125, 128)
    total = pl.pallas_call(
        _finalize_body,
        out_shape=jax.ShapeDtypeStruct((1, 1), jnp.float32),
    )(parts, pred_r, tgt_r)
    return total[0, 0]


# speedup vs baseline: 1.0349x; 1.0349x over previous
import functools

import jax
import jax.numpy as jnp
from jax import lax
from jax.experimental import pallas as pl
from jax.experimental.pallas import tpu as pltpu
from jax.experimental.pallas import tpu_sc as plsc

_N = 100000
_E = 3200000
_LAMBDA_CONT = 0.1
_LAMBDA_MOM = 0.01
_REYNOLDS = 1000000.0

_NW = 32
_SUB = 128
_NCHUNK = _E // _SUB
_CPW = 781
_KB = 71
_NB = 11
_NP = 100096
_RPT = _NP // 16
_TRB = 272


def _compute_chunk(rrows, crows, obuf):
    lane = lax.iota(jnp.int32, 16)
    for g in range(_SUB // 16):
        rid = lane + (g * 16)

        def ld(ref, k):
            return plsc.load_gather(
                ref, [rid, jnp.full((16,), k, jnp.int32)])

        rvx, rvy, rvz = ld(rrows, 0), ld(rrows, 1), ld(rrows, 2)
        rp = ld(rrows, 3)
        rpx, rpy, rpz = ld(rrows, 4), ld(rrows, 5), ld(rrows, 6)
        cvx, cvy, cvz = ld(crows, 0), ld(crows, 1), ld(crows, 2)
        cp = ld(crows, 3)
        cpx, cpy, cpz = ld(crows, 4), ld(crows, 5), ld(crows, 6)

        dx = cpx - rpx
        dy = cpy - rpy
        dz = cpz - rpz
        vdx = cvx - rvx
        vdy = cvy - rvy
        vdz = cvz - rvz
        pd = cp - rp

        d2 = dx * dx + dy * dy + dz * dz
        d2c = jnp.maximum(d2, 1.1754944e-38)
        bits = plsc.bitcast(d2c, jnp.int32)
        y = plsc.bitcast(
            jnp.full((16,), 0x5F3759DF, jnp.int32) - (bits >> 1),
            jnp.float32)
        h = 0.5 * d2c
        y = y * (1.5 - (h * y) * y)
        y = y * (1.5 - (h * y) * y)
        y = y * (1.5 - (h * y) * y)
        inv_dist = y - 1e-8 * (y * y)
        inv_d2e = 1.0 / (d2 + 1e-8)

        vg = (vdx * dx + vdy * dy + vdz * dz) * inv_dist
        wdx = vdx * inv_d2e
        wdy = vdy * inv_d2e
        wdz = vdz * inv_d2e
        pt = pd * inv_dist * inv_dist
        pgx = pt * dx
        pgy = pt * dy
        pgz = pt * dz
        ones = jnp.full((16,), 1.0, jnp.float32)

        for k, val in enumerate((vg, wdx, wdy, wdz, pgx, pgy, pgz,
                                 ones)):
            plsc.store_scatter(
                obuf, [rid, jnp.full((16,), k, jnp.int32)], val)


def _edge_body(tbl, ei_hbm, zrows, out_hbm,
               acc_sh, tbl_sh, ridx2_v, cidx2_v,
               rrows_a, crows_a, obuf_a, rrows_b, crows_b, obuf_b,
               rrows_c, crows_c, obuf_c,
               tin_v, tout2_v,
               gsem_a, gsem_b, gsem_c,
               ssem_a, ssem_b, ssem_c, tsem_a, tsem_b):
    cid = lax.axis_index("c")
    sid = lax.axis_index("s")
    wid = sid * 2 + cid
    wstart = _CPW * wid + jnp.minimum(wid, 8)

    pltpu.sync_copy(zrows, acc_sh.at[pl.ds(sid * _RPT, _RPT)])
    pltpu.sync_copy(tbl.at[pl.ds(sid * (_N // 16), _N // 16)],
                    tbl_sh.at[pl.ds(sid * (_N // 16), _N // 16)])
    plsc.subcore_barrier()

    def ridx(j):
        return ridx2_v.at[j]

    def cidx(j):
        return cidx2_v.at[j]

    def fire_gather(j, rrows, crows, gsem):
        gr = pltpu.async_copy(tbl_sh.at[ridx(j)], rrows, gsem)
        gc = pltpu.async_copy(tbl_sh.at[cidx(j)], crows, gsem)
        return gr, gc

    def wait_gather(j, rrows, crows, gsem):
        pltpu.make_async_copy(tbl_sh.at[ridx(j)], rrows, gsem).wait()
        pltpu.make_async_copy(tbl_sh.at[cidx(j)], crows, gsem).wait()

    def fire_scatter(j, obuf, ssem):
        return pltpu.async_copy(obuf, acc_sh.at[ridx(j)], ssem,
                                add=True)

    def wait_scatter(obuf, ssem):
        pltpu.make_async_copy(obuf, acc_sh.at[ridx(0)], ssem).wait()

    slots = ((rrows_a, crows_a, obuf_a, gsem_a, ssem_a),
             (rrows_b, crows_b, obuf_b, gsem_b, ssem_b),
             (rrows_c, crows_c, obuf_c, gsem_c, ssem_c))

    def block(b, _):
        start = wstart + b * _KB
        pltpu.sync_copy(ei_hbm.at[0, pl.ds(start, _KB)], ridx2_v)
        pltpu.sync_copy(ei_hbm.at[1, pl.ds(start, _KB)], cidx2_v)
        for s in range(3):
            rr, cr, ob, gs, ss = slots[s]
            fire_gather(s, rr, cr, gs)

        def triple(t, _):
            j0 = 3 * t
            for s in range(3):
                rr, cr, ob, gs, ss = slots[s]
                wait_gather(j0 + s, rr, cr, gs)

                @pl.when(t > 0)
                def _():
                    wait_scatter(ob, ss)

                _compute_chunk(rr, cr, ob)
                fire_scatter(j0 + s, ob, ss)

                @pl.when(j0 + s + 3 < _KB)
                def _():
                    fire_gather(j0 + s + 3, rr, cr, gs)
            return ()

        lax.fori_loop(0, (_KB - 2) // 3, triple, ())

        fin = []
        for s in range(2):
            rr, cr, ob, gs, ss = slots[s]
            j = _KB - 2 + s
            wait_gather(j, rr, cr, gs)
            wait_scatter(ob, ss)
            _compute_chunk(rr, cr, ob)
            fin.append(fire_scatter(j, ob, ss))
        wait_scatter(obuf_c, ssem_c)
        for f in fin:
            f.wait()
        return ()

    lax.fori_loop(0, _NB, block, ())

    @pl.when(wid < 8)
    def _():
        extra = wstart + _CPW
        pltpu.sync_copy(ei_hbm.at[0, pl.ds(extra, 1)],
                        ridx2_v.at[pl.ds(0, 1)])
        pltpu.sync_copy(ei_hbm.at[1, pl.ds(extra, 1)],
                        cidx2_v.at[pl.ds(0, 1)])
        gr, gc = fire_gather(0, rrows_a, crows_a, gsem_a)
        gr.wait()
        gc.wait()
        _compute_chunk(rrows_a, crows_a, obuf_a)
        pltpu.sync_copy(obuf_a, acc_sh.at[ridx(0)], add=True)

    plsc.subcore_barrier()

    lane = lax.iota(jnp.int32, 16)

    def trblock(t, _):
        local = t * _TRB
        pltpu.sync_copy(acc_sh.at[pl.ds(sid * _RPT + local, _TRB)], tin_v)
        for k in range(8):
            buf = tout2_v.at[k % 2]
            tsem = tsem_a if k % 2 == 0 else tsem_b
            if k >= 2:
                pltpu.make_async_copy(
                    buf, out_hbm.at[cid, sid, k - 2, pl.ds(local, _TRB)],
                    tsem).wait()
            kidx = jnp.full((16,), k, jnp.int32)
            for g in range(_TRB // 16):
                v = plsc.load_gather(tin_v, [lane + (g * 16), kidx])
                buf[pl.ds(g * 16, 16)] = v
            pltpu.async_copy(
                buf, out_hbm.at[cid, sid, k, pl.ds(local, _TRB)], tsem)
        for k in (6, 7):
            buf = tout2_v.at[k % 2]
            tsem = tsem_a if k % 2 == 0 else tsem_b
            pltpu.make_async_copy(
                buf, out_hbm.at[cid, sid, k, pl.ds(local, _TRB)],
                tsem).wait()
        return ()

    lax.fori_loop(0, _RPT // _TRB, trblock, ())


def _finalize_body(acc_ref, pred_ref, tgt_ref, out_ref):
    c_sum = jnp.float32(0.0)
    mom = jnp.float32(0.0)
    for t in range(16):
        a = acc_ref[0, t] + acc_ref[1, t]
        cnt = jnp.maximum(a[7:8, :], 1.0)
        inv_cnt = 1.0 / cnt
        div = a[0:1, :] * inv_cnt
        c_sum = c_sum + jnp.sum(div * div)
        for k in range(3):
            res = (a[1 + k:2 + k, :] * (1.0 / _REYNOLDS)
                   + a[4 + k:5 + k, :]) * inv_cnt
            mom = mom + jnp.sum(res * res)

    d = pred_ref[...] - tgt_ref[...]
    dsq = d * d
    col = lax.broadcasted_iota(jnp.int32, dsq.shape, 1)
    is_p = (col % 4) == 3
    p_sum = jnp.sum(jnp.where(is_p, dsq, 0.0))
    v_sum = jnp.sum(jnp.where(is_p, 0.0, dsq))

    total = (v_sum / (3.0 * _N) + p_sum / _N
             + _LAMBDA_CONT * (c_sum / _N)
             + _LAMBDA_MOM * (mom / (3.0 * _N)))
    out_ref[...] = jnp.reshape(total, (1, 1))


@jax.jit
def kernel(pred, target, edge_index, pos):
    tbl = jnp.concatenate(
        [pred, pos, jnp.zeros((_N, 1), jnp.float32)], axis=1)
    ei3 = edge_index.reshape(2, _NCHUNK, _SUB)
    zrows = jnp.zeros((_RPT, 8), jnp.float32)

    edge_fn = pl.kernel(
        _edge_body,
        out_type=jax.ShapeDtypeStruct((2, 16, 8, _RPT), jnp.float32),
        mesh=plsc.VectorSubcoreMesh(core_axis_name="c",
                                    subcore_axis_name="s"),
        scratch_types=[
            pltpu.VMEM_SHARED((_NP, 8), jnp.float32),
            pltpu.VMEM_SHARED((_N, 8), jnp.float32),
            pltpu.VMEM((_KB, _SUB), jnp.int32),
            pltpu.VMEM((_KB, _SUB), jnp.int32),
            pltpu.VMEM((_SUB, 8), jnp.float32),
            pltpu.VMEM((_SUB, 8), jnp.float32),
            pltpu.VMEM((_SUB, 8), jnp.float32),
            pltpu.VMEM((_SUB, 8), jnp.float32),
            pltpu.VMEM((_SUB, 8), jnp.float32),
            pltpu.VMEM((_SUB, 8), jnp.float32),
            pltpu.VMEM((_SUB, 8), jnp.float32),
            pltpu.VMEM((_SUB, 8), jnp.float32),
            pltpu.VMEM((_SUB, 8), jnp.float32),
            pltpu.VMEM((_TRB, 8), jnp.float32),
            pltpu.VMEM((2, _TRB), jnp.float32),
            pltpu.SemaphoreType.DMA,
            pltpu.SemaphoreType.DMA,
            pltpu.SemaphoreType.DMA,
            pltpu.SemaphoreType.DMA,
            pltpu.SemaphoreType.DMA,
            pltpu.SemaphoreType.DMA,
            pltpu.SemaphoreType.DMA,
            pltpu.SemaphoreType.DMA,
        ],
        compiler_params=pltpu.CompilerParams(
            needs_layout_passes=False,
            use_tc_tiling_on_sc=False,
            internal_scratch_in_bytes=1 << 20,
        ),
    )
    parts = edge_fn(tbl, ei3, zrows)

    pred_r = pred.reshape(3125, 128)
    tgt_r = target.reshape(3125, 128)
    total = pl.pallas_call(
        _finalize_body,
        out_shape=jax.ShapeDtypeStruct((1, 1), jnp.float32),
    )(parts, pred_r, tgt_r)
    return total[0, 0]


# speedup vs baseline: 1.1130x; 1.0755x over previous
import functools

import jax
import jax.numpy as jnp
from jax import lax
from jax.experimental import pallas as pl
from jax.experimental.pallas import tpu as pltpu
from jax.experimental.pallas import tpu_sc as plsc

_N = 100000
_E = 3200000
_LAMBDA_CONT = 0.1
_LAMBDA_MOM = 0.01
_REYNOLDS = 1000000.0

_NW = 32
_SUB = 128
_NCHUNK = _E // _SUB
_CPW = 781
_KB = 71
_NB = 11
_NP = 100096
_RPT = _NP // 16
_TRB = 272


def _compute_chunk(rrows, crows, obuf):
    lane = lax.iota(jnp.int32, 16)
    for g in range(_SUB // 16):
        rid = lane + (g * 16)

        def ld(ref, k):
            return plsc.load_gather(
                ref, [rid, jnp.full((16,), k, jnp.int32)])

        rvx, rvy, rvz = ld(rrows, 0), ld(rrows, 1), ld(rrows, 2)
        rp = ld(rrows, 3)
        rpx, rpy, rpz = ld(rrows, 4), ld(rrows, 5), ld(rrows, 6)
        cvx, cvy, cvz = ld(crows, 0), ld(crows, 1), ld(crows, 2)
        cp = ld(crows, 3)
        cpx, cpy, cpz = ld(crows, 4), ld(crows, 5), ld(crows, 6)

        dx = cpx - rpx
        dy = cpy - rpy
        dz = cpz - rpz
        vdx = cvx - rvx
        vdy = cvy - rvy
        vdz = cvz - rvz
        pd = cp - rp

        d2 = dx * dx + dy * dy + dz * dz
        d2c = jnp.maximum(d2, 1.1754944e-38)
        bits = plsc.bitcast(d2c, jnp.int32)
        y = plsc.bitcast(
            jnp.full((16,), 0x5F3759DF, jnp.int32) - (bits >> 1),
            jnp.float32)
        h = 0.5 * d2c
        y = y * (1.5 - (h * y) * y)
        y = y * (1.5 - (h * y) * y)
        inv_dist = y - 1e-8 * (y * y)
        inv_d2e = 1.0 / (d2 + 1e-8)

        vg = (vdx * dx + vdy * dy + vdz * dz) * inv_dist
        wdx = vdx * inv_d2e
        wdy = vdy * inv_d2e
        wdz = vdz * inv_d2e
        pt = pd * inv_dist * inv_dist
        pgx = pt * dx
        pgy = pt * dy
        pgz = pt * dz
        ones = jnp.full((16,), 1.0, jnp.float32)

        for k, val in enumerate((vg, wdx, wdy, wdz, pgx, pgy, pgz,
                                 ones)):
            plsc.store_scatter(
                obuf, [rid, jnp.full((16,), k, jnp.int32)], val)


def _edge_body(tbl, ei_hbm, zrows, out_hbm,
               acc_sh, tbl_sh, ridx2_v, cidx2_v,
               rrows_a, crows_a, obuf_a, rrows_b, crows_b, obuf_b,
               rrows_c, crows_c, obuf_c,
               tin_v, tout2_v,
               gsem_a, gsem_b, gsem_c,
               ssem_a, ssem_b, ssem_c, tsem_a, tsem_b):
    cid = lax.axis_index("c")
    sid = lax.axis_index("s")
    wid = sid * 2 + cid
    wstart = _CPW * wid + jnp.minimum(wid, 8)

    pltpu.sync_copy(zrows, acc_sh.at[pl.ds(sid * _RPT, _RPT)])
    pltpu.sync_copy(tbl.at[pl.ds(sid * (_N // 16), _N // 16)],
                    tbl_sh.at[pl.ds(sid * (_N // 16), _N // 16)])
    plsc.subcore_barrier()

    def ridx(j):
        return ridx2_v.at[j]

    def cidx(j):
        return cidx2_v.at[j]

    def fire_gather(j, rrows, crows, gsem):
        gr = pltpu.async_copy(tbl_sh.at[ridx(j)], rrows, gsem)
        gc = pltpu.async_copy(tbl_sh.at[cidx(j)], crows, gsem)
        return gr, gc

    def wait_gather(j, rrows, crows, gsem):
        pltpu.make_async_copy(tbl_sh.at[ridx(j)], rrows, gsem).wait()
        pltpu.make_async_copy(tbl_sh.at[cidx(j)], crows, gsem).wait()

    def fire_scatter(j, obuf, ssem):
        return pltpu.async_copy(obuf, acc_sh.at[ridx(j)], ssem,
                                add=True)

    def wait_scatter(obuf, ssem):
        pltpu.make_async_copy(obuf, acc_sh.at[ridx(0)], ssem).wait()

    slots = ((rrows_a, crows_a, obuf_a, gsem_a, ssem_a),
             (rrows_b, crows_b, obuf_b, gsem_b, ssem_b),
             (rrows_c, crows_c, obuf_c, gsem_c, ssem_c))

    def block(b, _):
        start = wstart + b * _KB
        pltpu.sync_copy(ei_hbm.at[0, pl.ds(start, _KB)], ridx2_v)
        pltpu.sync_copy(ei_hbm.at[1, pl.ds(start, _KB)], cidx2_v)
        for s in range(3):
            rr, cr, ob, gs, ss = slots[s]
            fire_gather(s, rr, cr, gs)

        def triple(t, _):
            j0 = 3 * t
            for s in range(3):
                rr, cr, ob, gs, ss = slots[s]
                wait_gather(j0 + s, rr, cr, gs)

                @pl.when(t > 0)
                def _():
                    wait_scatter(ob, ss)

                _compute_chunk(rr, cr, ob)
                fire_scatter(j0 + s, ob, ss)

                @pl.when(j0 + s + 3 < _KB)
                def _():
                    fire_gather(j0 + s + 3, rr, cr, gs)
            return ()

        lax.fori_loop(0, (_KB - 2) // 3, triple, ())

        fin = []
        for s in range(2):
            rr, cr, ob, gs, ss = slots[s]
            j = _KB - 2 + s
            wait_gather(j, rr, cr, gs)
            wait_scatter(ob, ss)
            _compute_chunk(rr, cr, ob)
            fin.append(fire_scatter(j, ob, ss))
        wait_scatter(obuf_c, ssem_c)
        for f in fin:
            f.wait()
        return ()

    lax.fori_loop(0, _NB, block, ())

    @pl.when(wid < 8)
    def _():
        extra = wstart + _CPW
        pltpu.sync_copy(ei_hbm.at[0, pl.ds(extra, 1)],
                        ridx2_v.at[pl.ds(0, 1)])
        pltpu.sync_copy(ei_hbm.at[1, pl.ds(extra, 1)],
                        cidx2_v.at[pl.ds(0, 1)])
        gr, gc = fire_gather(0, rrows_a, crows_a, gsem_a)
        gr.wait()
        gc.wait()
        _compute_chunk(rrows_a, crows_a, obuf_a)
        pltpu.sync_copy(obuf_a, acc_sh.at[ridx(0)], add=True)

    plsc.subcore_barrier()

    lane = lax.iota(jnp.int32, 16)

    def trblock(t, _):
        local = t * _TRB
        pltpu.sync_copy(acc_sh.at[pl.ds(sid * _RPT + local, _TRB)], tin_v)
        for k in range(8):
            buf = tout2_v.at[k % 2]
            tsem = tsem_a if k % 2 == 0 else tsem_b
            if k >= 2:
                pltpu.make_async_copy(
                    buf, out_hbm.at[cid, sid, k - 2, pl.ds(local, _TRB)],
                    tsem).wait()
            kidx = jnp.full((16,), k, jnp.int32)
            for g in range(_TRB // 16):
                v = plsc.load_gather(tin_v, [lane + (g * 16), kidx])
                buf[pl.ds(g * 16, 16)] = v
            pltpu.async_copy(
                buf, out_hbm.at[cid, sid, k, pl.ds(local, _TRB)], tsem)
        for k in (6, 7):
            buf = tout2_v.at[k % 2]
            tsem = tsem_a if k % 2 == 0 else tsem_b
            pltpu.make_async_copy(
                buf, out_hbm.at[cid, sid, k, pl.ds(local, _TRB)],
                tsem).wait()
        return ()

    lax.fori_loop(0, _RPT // _TRB, trblock, ())


def _finalize_body(acc_ref, pred_ref, tgt_ref, out_ref):
    c_sum = jnp.float32(0.0)
    mom = jnp.float32(0.0)
    for t in range(16):
        a = acc_ref[0, t] + acc_ref[1, t]
        cnt = jnp.maximum(a[7:8, :], 1.0)
        inv_cnt = 1.0 / cnt
        div = a[0:1, :] * inv_cnt
        c_sum = c_sum + jnp.sum(div * div)
        for k in range(3):
            res = (a[1 + k:2 + k, :] * (1.0 / _REYNOLDS)
                   + a[4 + k:5 + k, :]) * inv_cnt
            mom = mom + jnp.sum(res * res)

    d = pred_ref[...] - tgt_ref[...]
    dsq = d * d
    col = lax.broadcasted_iota(jnp.int32, dsq.shape, 1)
    is_p = (col % 4) == 3
    p_sum = jnp.sum(jnp.where(is_p, dsq, 0.0))
    v_sum = jnp.sum(jnp.where(is_p, 0.0, dsq))

    total = (v_sum / (3.0 * _N) + p_sum / _N
             + _LAMBDA_CONT * (c_sum / _N)
             + _LAMBDA_MOM * (mom / (3.0 * _N)))
    out_ref[...] = jnp.reshape(total, (1, 1))


@jax.jit
def kernel(pred, target, edge_index, pos):
    tbl = jnp.concatenate(
        [pred, pos, jnp.zeros((_N, 1), jnp.float32)], axis=1)
    ei3 = edge_index.reshape(2, _NCHUNK, _SUB)
    zrows = jnp.zeros((_RPT, 8), jnp.float32)

    edge_fn = pl.kernel(
        _edge_body,
        out_type=jax.ShapeDtypeStruct((2, 16, 8, _RPT), jnp.float32),
        mesh=plsc.VectorSubcoreMesh(core_axis_name="c",
                                    subcore_axis_name="s"),
        scratch_types=[
            pltpu.VMEM_SHARED((_NP, 8), jnp.float32),
            pltpu.VMEM_SHARED((_N, 8), jnp.float32),
            pltpu.VMEM((_KB, _SUB), jnp.int32),
            pltpu.VMEM((_KB, _SUB), jnp.int32),
            pltpu.VMEM((_SUB, 8), jnp.float32),
            pltpu.VMEM((_SUB, 8), jnp.float32),
            pltpu.VMEM((_SUB, 8), jnp.float32),
            pltpu.VMEM((_SUB, 8), jnp.float32),
            pltpu.VMEM((_SUB, 8), jnp.float32),
            pltpu.VMEM((_SUB, 8), jnp.float32),
            pltpu.VMEM((_SUB, 8), jnp.float32),
            pltpu.VMEM((_SUB, 8), jnp.float32),
            pltpu.VMEM((_SUB, 8), jnp.float32),
            pltpu.VMEM((_TRB, 8), jnp.float32),
            pltpu.VMEM((2, _TRB), jnp.float32),
            pltpu.SemaphoreType.DMA,
            pltpu.SemaphoreType.DMA,
            pltpu.SemaphoreType.DMA,
            pltpu.SemaphoreType.DMA,
            pltpu.SemaphoreType.DMA,
            pltpu.SemaphoreType.DMA,
            pltpu.SemaphoreType.DMA,
            pltpu.SemaphoreType.DMA,
        ],
        compiler_params=pltpu.CompilerParams(
            needs_layout_passes=False,
            use_tc_tiling_on_sc=False,
            internal_scratch_in_bytes=1 << 20,
        ),
    )
    parts = edge_fn(tbl, ei3, zrows)

    pred_r = pred.reshape(3125, 128)
    tgt_r = target.reshape(3125, 128)
    total = pl.pallas_call(
        _finalize_body,
        out_shape=jax.ShapeDtypeStruct((1, 1), jnp.float32),
    )(parts, pred_r, tgt_r)
    return total[0, 0]
